# Initial kernel scaffold; baseline (speedup 1.0000x reference)
import jax
import jax.numpy as jnp
from jax.experimental import pallas as pl


def kernel(table, input_):
    raise NotImplementedError("write your pallas kernel here")


# speedup vs baseline: 1.1027x; 1.1027x over previous
import functools

import jax
import jax.numpy as jnp
from jax import lax
from jax.experimental import pallas as pl
from jax.experimental.pallas import tpu as pltpu
from jax.experimental.pallas import tpu_sc as plsc

VOCAB = 1000000
DIM = 32
NC = 2
NS = 16
NW = NC * NS

CHUNK = 1600


def _make_gather(n_flat: int):
  assert n_flat % (NW * CHUNK) == 0
  per_w = n_flat // NW
  n_chunks = per_w // CHUNK

  mesh = plsc.VectorSubcoreMesh(
      core_axis_name="c", subcore_axis_name="s", num_cores=NC, num_subcores=NS
  )

  @functools.partial(
      pl.kernel,
      out_type=jax.ShapeDtypeStruct((n_flat, DIM), jnp.float32),
      mesh=mesh,
      scratch_types=[
          pltpu.VMEM((CHUNK,), jnp.int32),
          pltpu.VMEM((CHUNK, DIM), jnp.float32),
          pltpu.SemaphoreType.DMA,
      ],
      compiler_params=pltpu.CompilerParams(use_tc_tiling_on_sc=False),
  )
  def gather_kernel(table_hbm, idx_hbm, out_hbm, idx_v, rows_v, sem):
    wid = lax.axis_index("s") * NC + lax.axis_index("c")
    base = wid * per_w

    def body(i, carry):
      cbase = base + i * CHUNK
      pltpu.sync_copy(idx_hbm.at[pl.ds(cbase, CHUNK)], idx_v)
      pltpu.async_copy(table_hbm.at[idx_v], rows_v, sem).wait()
      pltpu.sync_copy(rows_v, out_hbm.at[pl.ds(cbase, CHUNK)])
      return carry

    lax.fori_loop(0, n_chunks, body, 0)

  return gather_kernel


@jax.jit
def kernel(table, input_):
  idx = input_.reshape(-1)
  out = _make_gather(idx.shape[0])(table, idx)
  return out.reshape(input_.shape + (DIM,))


# speedup vs baseline: 1.1120x; 1.0084x over previous
import functools

import jax
import jax.numpy as jnp
from jax import lax
from jax.experimental import pallas as pl
from jax.experimental.pallas import tpu as pltpu
from jax.experimental.pallas import tpu_sc as plsc

VOCAB = 1000000
DIM = 32
NC = 2
NS = 16
NW = NC * NS

NBUF = 2
CHUNK = 1600


def _make_gather(n_flat: int):
  assert n_flat % (NW * CHUNK * NBUF) == 0
  per_w = n_flat // NW
  n_chunks = per_w // CHUNK
  n_super = n_chunks // NBUF

  mesh = plsc.VectorSubcoreMesh(
      core_axis_name="c", subcore_axis_name="s", num_cores=NC, num_subcores=NS
  )

  @functools.partial(
      pl.kernel,
      out_type=jax.ShapeDtypeStruct((n_flat, DIM), jnp.float32),
      mesh=mesh,
      scratch_types=[
          [pltpu.VMEM((CHUNK,), jnp.int32) for _ in range(NBUF)],
          [pltpu.VMEM((CHUNK, DIM), jnp.float32) for _ in range(NBUF)],
          [pltpu.SemaphoreType.DMA for _ in range(NBUF)],
          [pltpu.SemaphoreType.DMA for _ in range(NBUF)],
      ],
      compiler_params=pltpu.CompilerParams(use_tc_tiling_on_sc=False),
  )
  def gather_kernel(table_hbm, idx_hbm, out_hbm, idxs, rows, sgs, sws):
    wid = lax.axis_index("s") * NC + lax.axis_index("c")
    base = wid * per_w

    for b in range(NBUF):
      pltpu.sync_copy(idx_hbm.at[pl.ds(base + b * CHUNK, CHUNK)], idxs[b])
      pltpu.async_copy(table_hbm.at[idxs[b]], rows[b], sgs[b])

    def body(s, carry):
      for b in range(NBUF):
        cbase = base + (s * NBUF + b) * CHUNK
        pltpu.make_async_copy(table_hbm.at[idxs[b]], rows[b], sgs[b]).wait()
        pltpu.async_copy(rows[b], out_hbm.at[pl.ds(cbase, CHUNK)], sws[b])

      @pl.when(s < n_super - 1)
      def _():
        for b in range(NBUF):
          nbase = base + ((s + 1) * NBUF + b) * CHUNK
          pltpu.sync_copy(idx_hbm.at[pl.ds(nbase, CHUNK)], idxs[b])
          pltpu.make_async_copy(
              rows[b], out_hbm.at[pl.ds(base, CHUNK)], sws[b]
          ).wait()
          pltpu.async_copy(table_hbm.at[idxs[b]], rows[b], sgs[b])

      return carry

    lax.fori_loop(0, n_super, body, 0)

    for b in range(NBUF):
      pltpu.make_async_copy(rows[b], out_hbm.at[pl.ds(base, CHUNK)], sws[b]).wait()

  return gather_kernel


@jax.jit
def kernel(table, input_):
  idx = input_.reshape(-1)
  out = _make_gather(idx.shape[0])(table, idx)
  return out.reshape(input_.shape + (DIM,))


# speedup vs baseline: 1.8059x; 1.6240x over previous
import functools

import jax
import jax.numpy as jnp
from jax import lax
from jax.experimental import pallas as pl
from jax.experimental.pallas import tpu as pltpu
from jax.experimental.pallas import tpu_sc as plsc

VOCAB = 1000000
DIM = 32
NC = 2
NS = 16
NW = NC * NS

NBUF = 2
CHUNK = 1600


def _make_gather(nb: int, nl: int):
  n_flat = nb * nl
  assert n_flat % (NW * CHUNK * NBUF) == 0 and CHUNK % nl == 0
  per_w = n_flat // NW
  n_chunks = per_w // CHUNK
  n_super = n_chunks // NBUF
  b_per_chunk = CHUNK // nl
  b_per_w = per_w // nl

  mesh = plsc.VectorSubcoreMesh(
      core_axis_name="c", subcore_axis_name="s", num_cores=NC, num_subcores=NS
  )

  @functools.partial(
      pl.kernel,
      out_type=jax.ShapeDtypeStruct((nb, nl, DIM), jnp.float32),
      mesh=mesh,
      scratch_types=[
          [pltpu.VMEM((CHUNK,), jnp.int32) for _ in range(NBUF)],
          [pltpu.VMEM((CHUNK, DIM), jnp.float32) for _ in range(NBUF)],
          [pltpu.SemaphoreType.DMA for _ in range(NBUF)],
          [pltpu.SemaphoreType.DMA for _ in range(NBUF)],
      ],
      compiler_params=pltpu.CompilerParams(use_tc_tiling_on_sc=False),
  )
  def gather_kernel(table_hbm, idx_hbm, out_hbm, idxs, rows, sgs, sws):
    wid = lax.axis_index("s") * NC + lax.axis_index("c")
    base = wid * per_w

    def fire_writebacks(b, bb):
      for k in range(b_per_chunk):
        pltpu.async_copy(
            rows[b].at[pl.ds(k * nl, nl)], out_hbm.at[bb + k], sws[b]
        )

    def drain_writebacks(b):
      for k in range(b_per_chunk):
        pltpu.make_async_copy(
            rows[b].at[pl.ds(k * nl, nl)], out_hbm.at[0], sws[b]
        ).wait()

    for b in range(NBUF):
      pltpu.sync_copy(idx_hbm.at[pl.ds(base + b * CHUNK, CHUNK)], idxs[b])
      pltpu.async_copy(table_hbm.at[idxs[b]], rows[b], sgs[b])

    def body(s, carry):
      for b in range(NBUF):
        i = s * NBUF + b
        bb = wid * b_per_w + i * b_per_chunk
        pltpu.make_async_copy(table_hbm.at[idxs[b]], rows[b], sgs[b]).wait()
        fire_writebacks(b, bb)

      @pl.when(s < n_super - 1)
      def _():
        for b in range(NBUF):
          nbase = base + ((s + 1) * NBUF + b) * CHUNK
          pltpu.sync_copy(idx_hbm.at[pl.ds(nbase, CHUNK)], idxs[b])
          drain_writebacks(b)
          pltpu.async_copy(table_hbm.at[idxs[b]], rows[b], sgs[b])

      return carry

    lax.fori_loop(0, n_super, body, 0)

    for b in range(NBUF):
      drain_writebacks(b)

  return gather_kernel


@jax.jit
def kernel(table, input_):
  nb, nl = input_.shape
  idx = input_.reshape(-1)
  return _make_gather(nb, nl)(table, idx)


# speedup vs baseline: 1.8061x; 1.0001x over previous
import functools

import jax
import jax.numpy as jnp
from jax import lax
from jax.experimental import pallas as pl
from jax.experimental.pallas import tpu as pltpu
from jax.experimental.pallas import tpu_sc as plsc

VOCAB = 1000000
DIM = 32
NC = 2
NS = 16
NW = NC * NS

NBUF = 2
CHUNK = 1600


def _make_gather(nb: int, nl: int):
  n_flat = nb * nl
  assert n_flat % (NW * CHUNK * NBUF) == 0 and CHUNK % nl == 0
  per_w = n_flat // NW
  n_chunks = per_w // CHUNK
  n_super = n_chunks // NBUF
  b_per_chunk = CHUNK // nl
  b_per_w = per_w // nl

  mesh = plsc.VectorSubcoreMesh(
      core_axis_name="c", subcore_axis_name="s", num_cores=NC, num_subcores=NS
  )

  @functools.partial(
      pl.kernel,
      out_type=jax.ShapeDtypeStruct((nb, nl, DIM), jnp.float32),
      mesh=mesh,
      scratch_types=[
          [pltpu.VMEM((CHUNK,), jnp.int32) for _ in range(NBUF)],
          [pltpu.VMEM((CHUNK, DIM), jnp.float32) for _ in range(NBUF)],
          [pltpu.SemaphoreType.DMA for _ in range(NBUF)],
          [pltpu.SemaphoreType.DMA for _ in range(NBUF)],
      ],
      compiler_params=pltpu.CompilerParams(use_tc_tiling_on_sc=False),
  )
  def gather_kernel(table_hbm, idx_hbm, out_hbm, idxs, rows, sgs, sws):
    wid = lax.axis_index("s") * NC + lax.axis_index("c")
    base = wid * per_w

    def fire_writebacks(b, bb):
      for k in range(b_per_chunk):
        pltpu.async_copy(
            rows[b].at[pl.ds(k * nl, nl)], out_hbm.at[bb + k], sws[b]
        )

    def drain_writebacks(b):
      for k in range(b_per_chunk):
        pltpu.make_async_copy(
            rows[b].at[pl.ds(k * nl, nl)], out_hbm.at[0], sws[b]
        ).wait()

    for b in range(NBUF):
      pltpu.sync_copy(idx_hbm.at[pl.ds(base + b * CHUNK, CHUNK)], idxs[b])
      pltpu.async_copy(table_hbm.at[idxs[b]], rows[b], sgs[b])

    def body(s, carry):
      for b in range(NBUF):
        i = s * NBUF + b
        bb = wid * b_per_w + i * b_per_chunk
        pltpu.make_async_copy(table_hbm.at[idxs[b]], rows[b], sgs[b]).wait()
        fire_writebacks(b, bb)

      @pl.when(s < n_super - 1)
      def _():
        for b in range(NBUF):
          nbase = base + ((s + 1) * NBUF + b) * CHUNK
          pltpu.sync_copy(idx_hbm.at[pl.ds(nbase, CHUNK)], idxs[b])
          drain_writebacks(b)
          pltpu.async_copy(table_hbm.at[idxs[b]], rows[b], sgs[b])

      return carry

    lax.fori_loop(0, n_super, body, 0)

    for b in range(NBUF):
      drain_writebacks(b)

  return gather_kernel


@jax.jit
def kernel(table, input_):
  nb, nl = input_.shape
  idx = jnp.reshape(input_ | 0, (-1,))
  return _make_gather(nb, nl)(table, idx)
